# baseline (speedup 1.0000x reference)
import functools

import jax
import jax.numpy as jnp
from jax import lax
from jax.experimental import pallas as pl
from jax.experimental.pallas import tpu as pltpu
from jax.experimental.pallas import tpu_sc as plsc

V = 1000
N_TOKENS = 16
HID = 256
B, S, L = 8, 4, 256
NW = 32
TPW = (B * S * L) // NW
ROWS_PER_SLAB = N_TOKENS + L
CHUNK = 32


def _learned_prompt(u, v, shared_prompt, mlp_w, mlp_b):
    def body(u_ref, v_ref, sp_ref, w_ref, b_ref, out_ref):
        learned = (u_ref[...] * v_ref[...]) * sp_ref[...]
        out_ref[...] = (
            jnp.dot(learned, w_ref[...], preferred_element_type=jnp.float32)
            + b_ref[...][None, :]
        )

    return pl.pallas_call(
        body,
        out_shape=jax.ShapeDtypeStruct((N_TOKENS, V), jnp.float32),
    )(u, v, shared_prompt, mlp_w, mlp_b)


def _sc_assemble(tokens_flat, wte, learned):
    mesh = plsc.VectorSubcoreMesh(core_axis_name="c", subcore_axis_name="s")
    n_chunks = TPW // CHUNK

    @functools.partial(
        pl.kernel,
        out_type=jax.ShapeDtypeStruct((B * S * ROWS_PER_SLAB, V), jnp.float32),
        mesh=mesh,
        compiler_params=pltpu.CompilerParams(use_tc_tiling_on_sc=False),
        scratch_types=[
            pltpu.VMEM((n_chunks, CHUNK), jnp.int32),
            pltpu.VMEM((CHUNK, V), jnp.float32),
            pltpu.VMEM((CHUNK, V), jnp.float32),
            pltpu.VMEM((N_TOKENS, V), jnp.float32),
            pltpu.SemaphoreType.DMA,
            pltpu.SemaphoreType.DMA,
        ],
    )
    def k(tok_hbm, wte_hbm, learned_hbm, out_hbm, idx_v, rows0, rows1,
          learned_v, sem0, sem1):
        wid = lax.axis_index("s") * 2 + lax.axis_index("c")
        out_base = wid * ROWS_PER_SLAB

        pltpu.sync_copy(tok_hbm.at[wid], idx_v)

        pltpu.sync_copy(learned_hbm, learned_v)
        pltpu.sync_copy(learned_v, out_hbm.at[pl.ds(out_base, N_TOKENS)])

        bufs = (rows0, rows1)
        sems = (sem0, sem1)
        copies = []
        for c in range(n_chunks):
            copies.append(
                pltpu.async_copy(wte_hbm.at[idx_v.at[c]], bufs[c % 2], sems[c % 2])
            )
            if c >= 1:
                copies[c - 1].wait()
                pltpu.sync_copy(
                    bufs[(c - 1) % 2],
                    out_hbm.at[pl.ds(out_base + N_TOKENS + (c - 1) * CHUNK, CHUNK)],
                )
        copies[n_chunks - 1].wait()
        pltpu.sync_copy(
            bufs[(n_chunks - 1) % 2],
            out_hbm.at[
                pl.ds(out_base + N_TOKENS + (n_chunks - 1) * CHUNK, CHUNK)
            ],
        )

    return k(tokens_flat, wte, learned)


def kernel(tokens, wte, mlp_w, mlp_b, shared_prompt, u, v):
    learned = _learned_prompt(u, v, shared_prompt, mlp_w, mlp_b)
    tokens_flat = tokens.reshape(NW, TPW // CHUNK, CHUNK).astype(jnp.int32)
    out = _sc_assemble(tokens_flat, wte, learned)
    return out.reshape(B, S, ROWS_PER_SLAB, V)


# speedup vs baseline: 1.7348x; 1.7348x over previous
import functools

import jax
import jax.numpy as jnp
from jax import lax
from jax.experimental import pallas as pl
from jax.experimental.pallas import tpu as pltpu
from jax.experimental.pallas import tpu_sc as plsc

V = 1000
VPAD = 1024
N_TOKENS = 16
HID = 256
B, S, L = 8, 4, 256
NW = 32
TPW = (B * S * L) // NW
ROWS_PER_SLAB = N_TOKENS + L
CHUNK = 32


def _learned_prompt(u, v, shared_prompt, mlp_w, mlp_b):
    def body(u_ref, v_ref, sp_ref, w_ref, b_ref, out_ref):
        learned = (u_ref[...] * v_ref[...]) * sp_ref[...]
        out_ref[...] = (
            jnp.dot(learned, w_ref[...], preferred_element_type=jnp.float32)
            + b_ref[...][None, :]
        )

    return pl.pallas_call(
        body,
        out_shape=jax.ShapeDtypeStruct((N_TOKENS, V), jnp.float32),
    )(u, v, shared_prompt, mlp_w, mlp_b)


def _sc_assemble(tokens_flat, wte_pad, learned):
    mesh = plsc.VectorSubcoreMesh(core_axis_name="c", subcore_axis_name="s")
    n_chunks = TPW // CHUNK

    @functools.partial(
        pl.kernel,
        out_type=jax.ShapeDtypeStruct((B * S * ROWS_PER_SLAB, V), jnp.float32),
        mesh=mesh,
        scratch_types=[
            pltpu.VMEM((n_chunks, CHUNK), jnp.int32),
            pltpu.VMEM((CHUNK, VPAD), jnp.float32),
            pltpu.VMEM((CHUNK, VPAD), jnp.float32),
            pltpu.VMEM((N_TOKENS, V), jnp.float32),
            pltpu.SemaphoreType.DMA,
            pltpu.SemaphoreType.DMA,
        ],
    )
    def k(tok_hbm, wte_hbm, learned_hbm, out_hbm, idx_v, rows0, rows1,
          learned_v, sem0, sem1):
        wid = lax.axis_index("s") * 2 + lax.axis_index("c")
        out_base = wid * ROWS_PER_SLAB

        pltpu.sync_copy(tok_hbm.at[wid], idx_v)

        pltpu.sync_copy(learned_hbm, learned_v)
        pltpu.sync_copy(learned_v, out_hbm.at[pl.ds(out_base, N_TOKENS)])

        def scatter_rows(buf, row_base):
            for t in range(VPAD // 128 - 1):
                w = 128 if (t + 1) * 128 <= V else V - t * 128
                pltpu.sync_copy(
                    buf.at[:, pl.ds(t * 128, w)],
                    out_hbm.at[pl.ds(row_base, CHUNK), pl.ds(t * 128, w)],
                )

        bufs = (rows0, rows1)
        sems = (sem0, sem1)
        copies = []
        for c in range(n_chunks):
            copies.append(
                pltpu.async_copy(wte_hbm.at[idx_v.at[c]], bufs[c % 2], sems[c % 2])
            )
            if c >= 1:
                copies[c - 1].wait()
                scatter_rows(bufs[(c - 1) % 2],
                             out_base + N_TOKENS + (c - 1) * CHUNK)
        copies[n_chunks - 1].wait()
        scatter_rows(bufs[(n_chunks - 1) % 2],
                     out_base + N_TOKENS + (n_chunks - 1) * CHUNK)

    return k(tokens_flat, wte_pad, learned)


def kernel(tokens, wte, mlp_w, mlp_b, shared_prompt, u, v):
    learned = _learned_prompt(u, v, shared_prompt, mlp_w, mlp_b)
    tokens_flat = tokens.reshape(NW, TPW // CHUNK, CHUNK).astype(jnp.int32)
    wte_pad = jnp.pad(wte, ((0, 0), (0, VPAD - V)))
    out = _sc_assemble(tokens_flat, wte_pad, learned)
    return out.reshape(B, S, ROWS_PER_SLAB, V)
